# baseline (speedup 1.0000x reference)
import jax
import jax.numpy as jnp
from jax import lax
from jax.experimental import pallas as pl
from jax.experimental.pallas import tpu as pltpu
from jax.experimental.pallas import tpu_sc as plsc

NUM_CLASSES = 26
HIDDEN = 128
LANES = 16

_info = plsc.get_sparse_core_info()
_NC, _NS = _info.num_cores, _info.num_subcores
_NW = _NC * _NS


def _sc_body(x_hbm, tab_hbm, out_hbm, tab_v, inp_v, out_v):
    tw = inp_v.shape[0]
    wid = lax.axis_index("s") * _NC + lax.axis_index("c")
    base = wid * tw
    pltpu.sync_copy(tab_hbm, tab_v)
    pltpu.sync_copy(x_hbm.at[pl.ds(base, tw)], inp_v)

    inv = jnp.float32(1.0 / NUM_CLASSES)
    for blk in range(HIDDEN // (2 * LANES)):
        lo = blk * 2 * LANES
        tab_a = [tab_v[c, pl.ds(lo, LANES)] * inv for c in range(NUM_CLASSES)]
        tab_b = [tab_v[c, pl.ds(lo + LANES, LANES)] * inv
                 for c in range(NUM_CLASSES)]

        def body(t, _, lo=lo, tab_a=tab_a, tab_b=tab_b):
            w0 = inp_v[t, pl.ds(0, LANES)]
            w1 = inp_v[t, pl.ds(NUM_CLASSES - LANES, LANES)]
            w = [w0[c] for c in range(LANES)]
            w += [w1[c - (NUM_CLASSES - LANES)] for c in range(LANES, NUM_CLASSES)]
            acc_a = w[0] * tab_a[0]
            acc_b = w[0] * tab_b[0]
            for c in range(1, NUM_CLASSES):
                acc_a += w[c] * tab_a[c]
                acc_b += w[c] * tab_b[c]
            out_v[t, pl.ds(lo, LANES)] = acc_a
            out_v[t, pl.ds(lo + LANES, LANES)] = acc_b
            return _

        lax.fori_loop(0, tw, body, None)

    pltpu.sync_copy(out_v, out_hbm.at[pl.ds(base, tw)])


def kernel(inputs, emb_table):
    B, L, C = inputs.shape
    M = B * L
    tw = M // _NW
    x = inputs.reshape(M, C)
    out = pl.kernel(
        _sc_body,
        out_type=jax.ShapeDtypeStruct((M, HIDDEN), jnp.float32),
        mesh=plsc.VectorSubcoreMesh(core_axis_name="c", subcore_axis_name="s"),
        compiler_params=pltpu.CompilerParams(use_tc_tiling_on_sc=False),
        scratch_types=[
            pltpu.VMEM((NUM_CLASSES, HIDDEN), jnp.float32),
            pltpu.VMEM((tw, NUM_CLASSES), jnp.float32),
            pltpu.VMEM((tw, HIDDEN), jnp.float32),
        ],
    )(x, emb_table)
    return out.reshape(B, L, HIDDEN)
